# baseline (speedup 1.0000x reference)
import functools

import jax
import jax.numpy as jnp
from jax import lax
from jax.experimental import pallas as pl
from jax.experimental.pallas import tpu as pltpu
from jax.experimental.pallas import tpu_sc as plsc

B, S, D, H, DH = 1, 2048, 1024, 16, 64
E, K, DFF = 8, 2, 4096

BQ = 512
BS = 512
FB = 1024

T = S * B
TK = T * K
BLK = 256
NPAD = TK + E * BLK
NB = NPAD // BLK
TRASH = TK

NC, NS, L = 2, 16, 16
NW = NC * NS
RPW = NPAD // NW
NCH = 3


def _qkv_body(x_ref, wq_ref, wk_ref, wv_ref, q_ref, k_ref, v_ref):
    x = x_ref[...]
    q_ref[0] = jnp.dot(x, wq_ref[0], preferred_element_type=jnp.float32)
    k_ref[0] = jnp.dot(x, wk_ref[0], preferred_element_type=jnp.float32)
    v_ref[0] = jnp.dot(x, wv_ref[0], preferred_element_type=jnp.float32)


def _attn_body(q_ref, k_ref, v_ref, o_ref):
    q = q_ref[0]
    k = k_ref[0]
    v = v_ref[0]
    s = jnp.dot(q, k.T, preferred_element_type=jnp.float32) * (1.0 / (DH ** 0.5))
    s = s - jnp.max(s, axis=-1, keepdims=True)
    p = jnp.exp(s)
    p = p / jnp.sum(p, axis=-1, keepdims=True)
    o_ref[0] = jnp.dot(p, v, preferred_element_type=jnp.float32)


def _proj_router_body(o_ref, x_ref, wo_ref, wr_ref, x1_ref, x1b_ref, eid_ref, w_ref):
    x1 = jnp.dot(o_ref[...], wo_ref[...], preferred_element_type=jnp.float32) + x_ref[...]
    x1_ref[...] = x1
    x1b_ref[...] = x1.astype(jnp.bfloat16)
    logits = jnp.dot(x1, wr_ref[...], preferred_element_type=jnp.float32)
    m = jnp.max(logits, axis=-1, keepdims=True)
    p = jnp.exp(logits - m)
    probs = p / jnp.sum(p, axis=-1, keepdims=True)
    lane = lax.broadcasted_iota(jnp.int32, probs.shape, 1)
    v0 = jnp.max(probs, axis=-1, keepdims=True)
    i0 = jnp.min(jnp.where(probs == v0, lane, E), axis=-1, keepdims=True)
    probs1 = jnp.where(lane == i0, -jnp.inf, probs)
    v1 = jnp.max(probs1, axis=-1, keepdims=True)
    i1 = jnp.min(jnp.where(probs1 == v1, lane, E), axis=-1, keepdims=True)
    denom = v0 + v1 + 1e-9
    eid_ref[...] = jnp.concatenate([i0, i1], axis=1)
    w_ref[...] = jnp.concatenate([v0 / denom, v1 / denom], axis=1)


def _ffn_body(blk_ref, xg_ref, w1_ref, b1_ref, w2_ref, b2_ref, out_ref, acc_ref):
    f = pl.program_id(0)
    b = pl.program_id(1)
    h = jnp.maximum(
        jnp.dot(xg_ref[...].astype(jnp.bfloat16),
                w1_ref[0].astype(jnp.bfloat16),
                preferred_element_type=jnp.float32)
        + b1_ref[0, 0], 0.0)
    part = jnp.dot(h.astype(jnp.bfloat16), w2_ref[0].astype(jnp.bfloat16),
                   preferred_element_type=jnp.float32)
    sl = pl.ds(b * BLK, BLK)

    @pl.when(f == 0)
    def _():
        acc_ref[sl, :] = part + b2_ref[0, 0]

    @pl.when(jnp.logical_and(f != 0, f != DFF // FB - 1))
    def _():
        acc_ref[sl, :] += part

    @pl.when(f == DFF // FB - 1)
    def _():
        out_ref[...] = acc_ref[sl, :] + part


def _combine_body(x1_ref, ys_ref, w_ref, out_ref):
    w = w_ref[...]
    ys = ys_ref[...]
    out_ref[...] = (x1_ref[...] + w[:, 0:1] * ys[:, :D]
                    + w[:, 1:2] * ys[:, D:])


_sc_mesh = plsc.VectorSubcoreMesh(core_axis_name="c", subcore_axis_name="s")


@functools.partial(
    pl.kernel,
    mesh=_sc_mesh,
    out_type=(
        jax.ShapeDtypeStruct((NPAD,), jnp.int32),
        jax.ShapeDtypeStruct((NPAD,), jnp.int32),
        jax.ShapeDtypeStruct((32,), jnp.int32),
    ),
    scratch_types=[
        pltpu.VMEM((TK,), jnp.int32),
        pltpu.VMEM((NPAD,), jnp.int32),
        pltpu.VMEM((NPAD,), jnp.int32),
        pltpu.VMEM((32,), jnp.int32),
        pltpu.VMEM((16,), jnp.int32),
    ],
    compiler_params=pltpu.CompilerParams(needs_layout_passes=False),
)
def _route_sort(eid_hbm, src_hbm, dst_hbm, blk_hbm,
                eid_v, src_v, dst_v, blk_v, cur_v):
    cid = lax.axis_index("c")
    sid = lax.axis_index("s")

    @pl.when(jnp.logical_and(cid == 0, sid == 0))
    def _():
        pltpu.sync_copy(eid_hbm, eid_v)
        lanes = lax.broadcasted_iota(jnp.int32, (L,), 0)

        def hist_step(i, cnt):
            ev = eid_v[pl.ds(i * L, L)]
            for e in range(E):
                c = jnp.sum((ev == e).astype(jnp.int32))
                cnt = cnt + jnp.where(lanes == e, c, 0)
            return cnt

        cnt = lax.fori_loop(0, TK // L, hist_step,
                            jnp.zeros((L,), jnp.int32))
        padded = ((cnt + (BLK - 1)) // BLK) * BLK
        ends = plsc.cumsum(padded)
        off = ends - padded
        cur_v[...] = off

        blk_v[pl.ds(0, L)] = jnp.zeros((L,), jnp.int32)
        blk_v[pl.ds(L, L)] = jnp.zeros((L,), jnp.int32)
        for b in range(NB):
            c = jnp.sum((b * BLK >= ends).astype(jnp.int32))
            be = jnp.minimum(c, E - 1)
            plsc.store_scatter(blk_v, [jnp.full((L,), b, jnp.int32)],
                               jnp.full((L,), be, jnp.int32),
                               mask=lanes == 0)

        def init_step(j, _):
            src_v[pl.ds(j * L, L)] = jnp.zeros((L,), jnp.int32)
            dst_v[pl.ds(j * L, L)] = jnp.full((L,), TRASH, jnp.int32)
            return 0

        lax.fori_loop(0, NPAD // L, init_step, 0)

        def sort_step(i, _):
            ev = eid_v[pl.ds(i * L, L)]
            iv = i * L + lanes
            base = plsc.load_gather(cur_v, [ev])
            rank = jnp.zeros((L,), jnp.int32)
            add = jnp.zeros((L,), jnp.int32)
            for e in range(E):
                m = ev == e
                pc = plsc.cumsum(m.astype(jnp.int32))
                rank = jnp.where(m, pc - 1, rank)
                add = add + jnp.where(lanes == e, jnp.max(pc), 0)
            pos = base + rank
            plsc.store_scatter(src_v, [pos], iv // K)
            plsc.store_scatter(dst_v, [pos], iv)
            cur_v[...] = cur_v[...] + add
            return 0

        lax.fori_loop(0, TK // L, sort_step, 0)

        pltpu.sync_copy(src_v, src_hbm)
        pltpu.sync_copy(dst_v, dst_hbm)
        pltpu.sync_copy(blk_v, blk_hbm)


RCHU = NPAD // NCH
RPWC = RCHU // NW


@functools.partial(
    pl.kernel,
    mesh=_sc_mesh,
    out_type=jax.ShapeDtypeStruct((RCHU, D), jnp.float32),
    scratch_types=[
        pltpu.VMEM((RPWC,), jnp.int32),
        pltpu.VMEM((RPWC, D), jnp.float32),
        pltpu.SemaphoreType.DMA,
    ],
    compiler_params=pltpu.CompilerParams(needs_layout_passes=False),
)
def _gather_rows(x1_hbm, src_hbm, xg_hbm, idx_v, buf, sem):
    wid = lax.axis_index("s") * NC + lax.axis_index("c")
    base = wid * RPWC
    pltpu.sync_copy(src_hbm.at[pl.ds(base, RPWC)], idx_v)
    pltpu.async_copy(x1_hbm.at[idx_v], buf, sem).wait()
    pltpu.sync_copy(buf, xg_hbm.at[pl.ds(base, RPWC)])


@functools.partial(
    pl.kernel,
    mesh=_sc_mesh,
    out_type=jax.ShapeDtypeStruct((TK + 8, D), jnp.float32),
    scratch_types=[
        pltpu.VMEM((4, RPW // 4), jnp.int32),
        pltpu.VMEM((RPW // 4, D), jnp.float32),
        pltpu.VMEM((RPW // 4, D), jnp.float32),
        pltpu.SemaphoreType.DMA,
        pltpu.SemaphoreType.DMA,
    ],
    compiler_params=pltpu.CompilerParams(needs_layout_passes=False),
)
def _scatter_rows(yp_hbm, dst_hbm, ys_hbm, idx_v, buf0, buf1, s0, s1):
    wid = lax.axis_index("s") * NC + lax.axis_index("c")
    base = wid * RPW
    qr = RPW // 4
    for c in range(4):
        pltpu.sync_copy(dst_hbm.at[pl.ds(base + c * qr, qr)], idx_v.at[c])
    bufs, sems = (buf0, buf1), (s0, s1)
    scats = [None] * 4
    for c in range(4):
        b = c % 2
        if c >= 2:
            scats[c - 2].wait()
        pltpu.sync_copy(yp_hbm.at[pl.ds(base + c * qr, qr)], bufs[b])
        scats[c] = pltpu.async_copy(
            bufs[b], ys_hbm.at[idx_v.at[c]], sems[b])
    scats[2].wait()
    scats[3].wait()


def kernel(x, Wq, Wk, Wv, Wo, Wr, W1, b1, W2, b2):
    xf = x.reshape(S, D)
    wq_h = Wq.reshape(D, H, DH).transpose(1, 0, 2)
    wk_h = Wk.reshape(D, H, DH).transpose(1, 0, 2)
    wv_h = Wv.reshape(D, H, DH).transpose(1, 0, 2)
    b1_3 = b1.reshape(E, 1, DFF)
    b2_3 = b2.reshape(E, 1, D)

    q, k, v = pl.pallas_call(
        _qkv_body,
        grid=(H,),
        in_specs=[
            pl.BlockSpec((S, D), lambda h: (0, 0)),
            pl.BlockSpec((1, D, DH), lambda h: (h, 0, 0)),
            pl.BlockSpec((1, D, DH), lambda h: (h, 0, 0)),
            pl.BlockSpec((1, D, DH), lambda h: (h, 0, 0)),
        ],
        out_specs=[
            pl.BlockSpec((1, S, DH), lambda h: (h, 0, 0)),
            pl.BlockSpec((1, S, DH), lambda h: (h, 0, 0)),
            pl.BlockSpec((1, S, DH), lambda h: (h, 0, 0)),
        ],
        out_shape=[jax.ShapeDtypeStruct((H, S, DH), jnp.float32)] * 3,
    )(xf, wq_h, wk_h, wv_h)

    o_h = pl.pallas_call(
        _attn_body,
        grid=(H, S // BQ),
        in_specs=[
            pl.BlockSpec((1, BQ, DH), lambda h, s: (h, s, 0)),
            pl.BlockSpec((1, S, DH), lambda h, s: (h, 0, 0)),
            pl.BlockSpec((1, S, DH), lambda h, s: (h, 0, 0)),
        ],
        out_specs=pl.BlockSpec((1, BQ, DH), lambda h, s: (h, s, 0)),
        out_shape=jax.ShapeDtypeStruct((H, S, DH), jnp.float32),
    )(q, k, v)
    o = o_h.transpose(1, 0, 2).reshape(S, D)

    x1, x1b, eid, w = pl.pallas_call(
        _proj_router_body,
        grid=(S // BS,),
        in_specs=[
            pl.BlockSpec((BS, D), lambda s: (s, 0)),
            pl.BlockSpec((BS, D), lambda s: (s, 0)),
            pl.BlockSpec((D, D), lambda s: (0, 0)),
            pl.BlockSpec((D, E), lambda s: (0, 0)),
        ],
        out_specs=[
            pl.BlockSpec((BS, D), lambda s: (s, 0)),
            pl.BlockSpec((BS, D), lambda s: (s, 0)),
            pl.BlockSpec((BS, K), lambda s: (s, 0)),
            pl.BlockSpec((BS, K), lambda s: (s, 0)),
        ],
        out_shape=[
            jax.ShapeDtypeStruct((S, D), jnp.float32),
            jax.ShapeDtypeStruct((S, D), jnp.bfloat16),
            jax.ShapeDtypeStruct((T, K), jnp.int32),
            jax.ShapeDtypeStruct((T, K), jnp.float32),
        ],
    )(o, xf, Wo, Wr)

    src, dst, blk_e = _route_sort(eid.reshape(TK))

    yp_chunks = []
    for ci in range(NCH):
        xg_c = _gather_rows(x1, lax.slice(src, (ci * RCHU,),
                                          ((ci + 1) * RCHU,)))
        nbc = RCHU // BLK
        yp_c = pl.pallas_call(
            _ffn_body,
            grid_spec=pltpu.PrefetchScalarGridSpec(
                num_scalar_prefetch=1,
                grid=(DFF // FB, nbc),
                in_specs=[
                    pl.BlockSpec((BLK, D), lambda f, b, blk: (b, 0)),
                    pl.BlockSpec((1, D, FB),
                                 lambda f, b, blk, c0=ci * nbc: (blk[b + c0], 0, f)),
                    pl.BlockSpec((1, 1, FB),
                                 lambda f, b, blk, c0=ci * nbc: (blk[b + c0], 0, f)),
                    pl.BlockSpec((1, FB, D),
                                 lambda f, b, blk, c0=ci * nbc: (blk[b + c0], f, 0)),
                    pl.BlockSpec((1, 1, D),
                                 lambda f, b, blk, c0=ci * nbc: (blk[b + c0], 0, 0)),
                ],
                out_specs=pl.BlockSpec((BLK, D), lambda f, b, blk: (b, 0)),
                scratch_shapes=[pltpu.VMEM((RCHU, D), jnp.float32)],
            ),
            out_shape=jax.ShapeDtypeStruct((RCHU, D), jnp.float32),
        )(blk_e, xg_c, W1, b1_3, W2, b2_3)
        yp_chunks.append(yp_c)

    yp = jnp.concatenate(yp_chunks, axis=0)
    ys = _scatter_rows(yp, dst)
    ys2 = ys[:TK].reshape(T, K * D)

    out = pl.pallas_call(
        _combine_body,
        grid=(S // BS,),
        in_specs=[
            pl.BlockSpec((BS, D), lambda s: (s, 0)),
            pl.BlockSpec((BS, K * D), lambda s: (s, 0)),
            pl.BlockSpec((BS, K), lambda s: (s, 0)),
        ],
        out_specs=pl.BlockSpec((BS, D), lambda s: (s, 0)),
        out_shape=jax.ShapeDtypeStruct((S, D), jnp.float32),
    )(x1, ys2, w)

    return out.reshape(B, S, D)


# speedup vs baseline: 1.1779x; 1.1779x over previous
import jax
import jax.numpy as jnp
from jax import lax
from jax.experimental import pallas as pl
from jax.experimental.pallas import tpu as pltpu

B, S, D, H, DH = 1, 2048, 1024, 16, 64
E, K, DFF = 8, 2, 4096

BQ = 512
BS = 512
FB = 1024


def _qkv_body(x_ref, wq_ref, wk_ref, wv_ref, q_ref, k_ref, v_ref):
    x = x_ref[...]
    q_ref[0] = jnp.dot(x, wq_ref[0], preferred_element_type=jnp.float32)
    k_ref[0] = jnp.dot(x, wk_ref[0], preferred_element_type=jnp.float32)
    v_ref[0] = jnp.dot(x, wv_ref[0], preferred_element_type=jnp.float32)


def _attn_body(q_ref, k_ref, v_ref, o_ref):
    q = q_ref[0]
    k = k_ref[0]
    v = v_ref[0]
    s = jnp.dot(q, k.T, preferred_element_type=jnp.float32) * (1.0 / (DH ** 0.5))
    s = s - jnp.max(s, axis=-1, keepdims=True)
    p = jnp.exp(s)
    p = p / jnp.sum(p, axis=-1, keepdims=True)
    o_ref[0] = jnp.dot(p, v, preferred_element_type=jnp.float32)


def _proj_router_body(o_ref, x_ref, wo_ref, wr_ref, x1_ref, gate_ref):
    x1 = jnp.dot(o_ref[...], wo_ref[...], preferred_element_type=jnp.float32) + x_ref[...]
    x1_ref[...] = x1
    logits = jnp.dot(x1, wr_ref[...], preferred_element_type=jnp.float32)
    m = jnp.max(logits, axis=-1, keepdims=True)
    p = jnp.exp(logits - m)
    probs = p / jnp.sum(p, axis=-1, keepdims=True)
    lane = lax.broadcasted_iota(jnp.int32, probs.shape, 1)
    v0 = jnp.max(probs, axis=-1, keepdims=True)
    i0 = jnp.min(jnp.where(probs == v0, lane, E), axis=-1, keepdims=True)
    probs1 = jnp.where(lane == i0, -jnp.inf, probs)
    v1 = jnp.max(probs1, axis=-1, keepdims=True)
    i1 = jnp.min(jnp.where(probs1 == v1, lane, E), axis=-1, keepdims=True)
    denom = v0 + v1 + 1e-9
    w0 = v0 / denom
    w1 = v1 / denom
    gate_ref[...] = jnp.where(lane == i0, w0, 0.0) + jnp.where(lane == i1, w1, 0.0)


def _moe_body(x1_ref, gate_ref, w1_ref, b1_ref, w2_ref, b2_ref, out_ref,
              acc_ref):
    e = pl.program_id(0)
    f = pl.program_id(1)
    sblk = pl.program_id(2)
    sl = pl.ds(sblk * BS, BS)
    x1 = x1_ref[sl, :]
    lane = lax.broadcasted_iota(jnp.int32, (BS, E), 1)
    g = jnp.sum(jnp.where(lane == e, gate_ref[sl, :], 0.0), axis=-1,
                keepdims=True)
    h = jnp.maximum(
        jnp.dot(x1.astype(jnp.bfloat16), w1_ref[0].astype(jnp.bfloat16),
                preferred_element_type=jnp.float32) + b1_ref[0, 0], 0.0)
    part = jnp.dot(h.astype(jnp.bfloat16), w2_ref[0].astype(jnp.bfloat16),
                   preferred_element_type=jnp.float32)

    @pl.when(jnp.logical_and(e == 0, f == 0))
    def _():
        acc_ref[sl, :] = x1

    @pl.when(f == 0)
    def _():
        acc_ref[sl, :] += g * b2_ref[0, 0]

    acc_ref[sl, :] += g * part

    @pl.when(jnp.logical_and(e == E - 1, f == DFF // FB - 1))
    def _():
        out_ref[...] = acc_ref[sl, :]


def kernel(x, Wq, Wk, Wv, Wo, Wr, W1, b1, W2, b2):
    xf = x.reshape(S, D)
    wq_h = Wq.reshape(D, H, DH).transpose(1, 0, 2)
    wk_h = Wk.reshape(D, H, DH).transpose(1, 0, 2)
    wv_h = Wv.reshape(D, H, DH).transpose(1, 0, 2)
    b1_3 = b1.reshape(E, 1, DFF)
    b2_3 = b2.reshape(E, 1, D)

    q, k, v = pl.pallas_call(
        _qkv_body,
        grid=(H,),
        in_specs=[
            pl.BlockSpec((S, D), lambda h: (0, 0)),
            pl.BlockSpec((1, D, DH), lambda h: (h, 0, 0)),
            pl.BlockSpec((1, D, DH), lambda h: (h, 0, 0)),
            pl.BlockSpec((1, D, DH), lambda h: (h, 0, 0)),
        ],
        out_specs=[
            pl.BlockSpec((1, S, DH), lambda h: (h, 0, 0)),
            pl.BlockSpec((1, S, DH), lambda h: (h, 0, 0)),
            pl.BlockSpec((1, S, DH), lambda h: (h, 0, 0)),
        ],
        out_shape=[jax.ShapeDtypeStruct((H, S, DH), jnp.float32)] * 3,
    )(xf, wq_h, wk_h, wv_h)

    o_h = pl.pallas_call(
        _attn_body,
        grid=(H, S // BQ),
        in_specs=[
            pl.BlockSpec((1, BQ, DH), lambda h, s: (h, s, 0)),
            pl.BlockSpec((1, S, DH), lambda h, s: (h, 0, 0)),
            pl.BlockSpec((1, S, DH), lambda h, s: (h, 0, 0)),
        ],
        out_specs=pl.BlockSpec((1, BQ, DH), lambda h, s: (h, s, 0)),
        out_shape=jax.ShapeDtypeStruct((H, S, DH), jnp.float32),
    )(q, k, v)
    o = o_h.transpose(1, 0, 2).reshape(S, D)

    x1, gate = pl.pallas_call(
        _proj_router_body,
        grid=(S // BS,),
        in_specs=[
            pl.BlockSpec((BS, D), lambda s: (s, 0)),
            pl.BlockSpec((BS, D), lambda s: (s, 0)),
            pl.BlockSpec((D, D), lambda s: (0, 0)),
            pl.BlockSpec((D, E), lambda s: (0, 0)),
        ],
        out_specs=[
            pl.BlockSpec((BS, D), lambda s: (s, 0)),
            pl.BlockSpec((BS, E), lambda s: (s, 0)),
        ],
        out_shape=[
            jax.ShapeDtypeStruct((S, D), jnp.float32),
            jax.ShapeDtypeStruct((S, E), jnp.float32),
        ],
    )(o, xf, Wo, Wr)

    out = pl.pallas_call(
        _moe_body,
        grid=(E, DFF // FB, S // BS),
        in_specs=[
            pl.BlockSpec((S, D), lambda e, f, s: (0, 0)),
            pl.BlockSpec((S, E), lambda e, f, s: (0, 0)),
            pl.BlockSpec((1, D, FB), lambda e, f, s: (e, 0, f)),
            pl.BlockSpec((1, 1, FB), lambda e, f, s: (e, 0, f)),
            pl.BlockSpec((1, FB, D), lambda e, f, s: (e, f, 0)),
            pl.BlockSpec((1, 1, D), lambda e, f, s: (e, 0, 0)),
        ],
        out_specs=pl.BlockSpec((BS, D), lambda e, f, s: (s, 0)),
        out_shape=jax.ShapeDtypeStruct((S, D), jnp.float32),
        scratch_shapes=[pltpu.VMEM((S, D), jnp.float32)],
    )(x1, gate, W1, b1_3, W2, b2_3)

    return out.reshape(B, S, D)


# speedup vs baseline: 1.2011x; 1.0197x over previous
import functools

import jax
import jax.numpy as jnp
from jax.experimental import pallas as pl

B, S, D, H, DH = 1, 2048, 1024, 16, 64
E, K, DFF = 8, 2, 4096

BQ = 512
BS = 512
FB = 1024


def _qkv_body(x_ref, wq_ref, wk_ref, wv_ref, q_ref, k_ref, v_ref):
    x = x_ref[...]
    q_ref[0] = jnp.dot(x, wq_ref[0], preferred_element_type=jnp.float32)
    k_ref[0] = jnp.dot(x, wk_ref[0], preferred_element_type=jnp.float32)
    v_ref[0] = jnp.dot(x, wv_ref[0], preferred_element_type=jnp.float32)


def _attn_body(q_ref, k_ref, v_ref, o_ref):
    q = q_ref[0]
    k = k_ref[0]
    v = v_ref[0]
    s = jnp.dot(q, k.T, preferred_element_type=jnp.float32) * (1.0 / (DH ** 0.5))
    s = s - jnp.max(s, axis=-1, keepdims=True)
    p = jnp.exp(s)
    p = p / jnp.sum(p, axis=-1, keepdims=True)
    o_ref[0] = jnp.dot(p, v, preferred_element_type=jnp.float32)


def _proj_router_body(o_ref, x_ref, wo_ref, wr_ref, x1_ref, gate_ref):
    x1 = jnp.dot(o_ref[...], wo_ref[...], preferred_element_type=jnp.float32) + x_ref[...]
    x1_ref[...] = x1
    logits = jnp.dot(x1, wr_ref[...], preferred_element_type=jnp.float32)
    m = jnp.max(logits, axis=-1, keepdims=True)
    p = jnp.exp(logits - m)
    probs = p / jnp.sum(p, axis=-1, keepdims=True)
    lane = jax.lax.broadcasted_iota(jnp.int32, probs.shape, 1)
    v0 = jnp.max(probs, axis=-1, keepdims=True)
    i0 = jnp.min(jnp.where(probs == v0, lane, E), axis=-1, keepdims=True)
    probs1 = jnp.where(lane == i0, -jnp.inf, probs)
    v1 = jnp.max(probs1, axis=-1, keepdims=True)
    i1 = jnp.min(jnp.where(probs1 == v1, lane, E), axis=-1, keepdims=True)
    denom = v0 + v1 + 1e-9
    w0 = v0 / denom
    w1 = v1 / denom
    gate_ref[...] = jnp.where(lane == i0, w0, 0.0) + jnp.where(lane == i1, w1, 0.0)


def _moe_body(x1_ref, gate_ref, w1_ref, b1_ref, w2_ref, b2_ref, out_ref):
    e = pl.program_id(1)
    f = pl.program_id(2)
    x1 = x1_ref[...]
    lane = jax.lax.broadcasted_iota(jnp.int32, gate_ref.shape, 1)
    g = jnp.sum(jnp.where(lane == e, gate_ref[...], 0.0), axis=-1,
                keepdims=True)
    h = jnp.maximum(jnp.dot(x1, w1_ref[0], preferred_element_type=jnp.float32)
                    + b1_ref[0, 0], 0.0)
    acc = jnp.dot(h, w2_ref[0], preferred_element_type=jnp.float32)

    @pl.when(jnp.logical_and(e == 0, f == 0))
    def _():
        out_ref[...] = x1

    @pl.when(f == 0)
    def _():
        out_ref[...] += g * b2_ref[0, 0]

    out_ref[...] += g * acc


def kernel(x, Wq, Wk, Wv, Wo, Wr, W1, b1, W2, b2):
    xf = x.reshape(S, D)
    wq_h = Wq.reshape(D, H, DH).transpose(1, 0, 2)
    wk_h = Wk.reshape(D, H, DH).transpose(1, 0, 2)
    wv_h = Wv.reshape(D, H, DH).transpose(1, 0, 2)
    b1_3 = b1.reshape(E, 1, DFF)
    b2_3 = b2.reshape(E, 1, D)

    q, k, v = pl.pallas_call(
        _qkv_body,
        grid=(H,),
        in_specs=[
            pl.BlockSpec((S, D), lambda h: (0, 0)),
            pl.BlockSpec((1, D, DH), lambda h: (h, 0, 0)),
            pl.BlockSpec((1, D, DH), lambda h: (h, 0, 0)),
            pl.BlockSpec((1, D, DH), lambda h: (h, 0, 0)),
        ],
        out_specs=[
            pl.BlockSpec((1, S, DH), lambda h: (h, 0, 0)),
            pl.BlockSpec((1, S, DH), lambda h: (h, 0, 0)),
            pl.BlockSpec((1, S, DH), lambda h: (h, 0, 0)),
        ],
        out_shape=[jax.ShapeDtypeStruct((H, S, DH), jnp.float32)] * 3,
    )(xf, wq_h, wk_h, wv_h)

    o_h = pl.pallas_call(
        _attn_body,
        grid=(H, S // BQ),
        in_specs=[
            pl.BlockSpec((1, BQ, DH), lambda h, s: (h, s, 0)),
            pl.BlockSpec((1, S, DH), lambda h, s: (h, 0, 0)),
            pl.BlockSpec((1, S, DH), lambda h, s: (h, 0, 0)),
        ],
        out_specs=pl.BlockSpec((1, BQ, DH), lambda h, s: (h, s, 0)),
        out_shape=jax.ShapeDtypeStruct((H, S, DH), jnp.float32),
    )(q, k, v)
    o = o_h.transpose(1, 0, 2).reshape(S, D)

    x1, gate = pl.pallas_call(
        _proj_router_body,
        grid=(S // BS,),
        in_specs=[
            pl.BlockSpec((BS, D), lambda s: (s, 0)),
            pl.BlockSpec((BS, D), lambda s: (s, 0)),
            pl.BlockSpec((D, D), lambda s: (0, 0)),
            pl.BlockSpec((D, E), lambda s: (0, 0)),
        ],
        out_specs=[
            pl.BlockSpec((BS, D), lambda s: (s, 0)),
            pl.BlockSpec((BS, E), lambda s: (s, 0)),
        ],
        out_shape=[
            jax.ShapeDtypeStruct((S, D), jnp.float32),
            jax.ShapeDtypeStruct((S, E), jnp.float32),
        ],
    )(o, xf, Wo, Wr)

    out = pl.pallas_call(
        _moe_body,
        grid=(S // BS, E, DFF // FB),
        in_specs=[
            pl.BlockSpec((BS, D), lambda s, e, f: (s, 0)),
            pl.BlockSpec((BS, E), lambda s, e, f: (s, 0)),
            pl.BlockSpec((1, D, FB), lambda s, e, f: (e, 0, f)),
            pl.BlockSpec((1, 1, FB), lambda s, e, f: (e, 0, f)),
            pl.BlockSpec((1, FB, D), lambda s, e, f: (e, f, 0)),
            pl.BlockSpec((1, 1, D), lambda s, e, f: (e, 0, 0)),
        ],
        out_specs=pl.BlockSpec((BS, D), lambda s, e, f: (s, 0)),
        out_shape=jax.ShapeDtypeStruct((S, D), jnp.float32),
    )(x1, gate, W1, b1_3, W2, b2_3)

    return out.reshape(B, S, D)
